# baseline (speedup 1.0000x reference)
import jax
import jax.numpy as jnp
import numpy as np
from jax.experimental import pallas as pl
from jax.experimental.pallas import tpu as pltpu

_GX = _GY = 64
_VOX = (1.6, 1.6, 0.6)
_PC_LO = (-51.2, -51.2, -3.0)
_D = 256
_HEADS = 8
_EPS = 1e-5

_NPTS = 120000
_PCHUNK = 1000
_NCHUNK = _NPTS // _PCHUNK
_HALF = _NCHUNK // 2


def _conv2d(x, w, b=None, stride=1, pad=0):
    y = jax.lax.conv_general_dilated(x, w, (stride, stride), ((pad, pad), (pad, pad)),
                                     dimension_numbers=("NCHW", "OIHW", "NCHW"))
    if b is not None:
        y = y + b[None, :, None, None]
    return y


def _bn2d(x, g, b):
    m = x.mean((0, 2, 3), keepdims=True)
    v = x.var((0, 2, 3), keepdims=True)
    return g[None, :, None, None] * (x - m) / jnp.sqrt(v + _EPS) + b[None, :, None, None]


def _layernorm(x, g, b):
    m = x.mean(-1, keepdims=True)
    v = x.var(-1, keepdims=True)
    return g * (x - m) / jnp.sqrt(v + _EPS) + b


def _maxpool3x3s2(x):
    return jax.lax.reduce_window(x, -jnp.inf, jax.lax.max, (1, 1, 3, 3), (1, 1, 2, 2),
                                 ((0, 0), (0, 0), (1, 1), (1, 1)))


def _basic_block(x, bp, stride):
    y = jax.nn.relu(_bn2d(_conv2d(x, bp["w1"], stride=stride, pad=1), bp["g1"], bp["b1"]))
    y = _bn2d(_conv2d(y, bp["w2"], stride=1, pad=1), bp["g2"], bp["b2"])
    if "wd" in bp:
        idn = _bn2d(_conv2d(x, bp["wd"], stride=stride), bp["gd"], bp["bd"])
    else:
        idn = x
    return jax.nn.relu(y + idn)


def _resnet34(x, p):
    x = jax.nn.relu(_bn2d(_conv2d(x, p["stem"]["w"], stride=2, pad=3), p["stem"]["g"], p["stem"]["b"]))
    x = _maxpool3x3s2(x)
    for stage, s0 in zip(p["stages"], (1, 2, 2, 2)):
        for i, bp in enumerate(stage):
            x = _basic_block(x, bp, s0 if i == 0 else 1)
    return x


def _mha(q, kv, p):
    B, Sq, Dm = q.shape
    h = _HEADS
    dh = Dm // h
    qp = q @ p["in_w"][:Dm].T + p["in_b"][:Dm]
    kp = kv @ p["in_w"][Dm:2 * Dm].T + p["in_b"][Dm:2 * Dm]
    vp = kv @ p["in_w"][2 * Dm:].T + p["in_b"][2 * Dm:]
    sp = lambda t: t.reshape(B, -1, h, dh).transpose(0, 2, 1, 3)
    qh, kh, vh = sp(qp), sp(kp), sp(vp)
    a = jax.nn.softmax(jnp.einsum("bhqd,bhkd->bhqk", qh, kh) / np.sqrt(dh).astype(np.float32), axis=-1)
    o = jnp.einsum("bhqk,bhkd->bhqd", a, vh).transpose(0, 2, 1, 3).reshape(B, Sq, Dm)
    return o @ p["out_w"].T + p["out_b"]


def _cross_layer(q, kv, p):
    x = _layernorm(q + _mha(q, kv, p), p["ln1g"], p["ln1b"])
    f = jax.nn.relu(x @ p["f1w"] + p["f1b"]) @ p["f2w"] + p["f2b"]
    return _layernorm(x + f, p["ln2g"], p["ln2b"])


def _stats1_kernel(pts_ref, w_ref, b_ref, out_ref):
    j = pl.program_id(0)

    @pl.when(j == 0)
    def _():
        out_ref[...] = jnp.zeros_like(out_ref)

    h = jnp.dot(pts_ref[...], w_ref[...], preferred_element_type=jnp.float32) + b_ref[...]
    s = jnp.sum(h, axis=0, keepdims=True)
    sq = jnp.sum(h * h, axis=0, keepdims=True)
    out_ref[...] += jnp.concatenate([s, sq], axis=0)


def _stats2_kernel(pts_ref, w1_ref, b1_ref, w2_ref, b2_ref, out_ref):
    j = pl.program_id(0)

    @pl.when(j == 0)
    def _():
        out_ref[...] = jnp.zeros_like(out_ref)

    h1 = jnp.maximum(jnp.dot(pts_ref[...], w1_ref[...], preferred_element_type=jnp.float32)
                     + b1_ref[...], 0.0)
    h2 = jnp.dot(h1, w2_ref[...], preferred_element_type=jnp.float32) + b2_ref[...]
    s = jnp.sum(h2, axis=0, keepdims=True)
    sq = jnp.sum(h2 * h2, axis=0, keepdims=True)
    out_ref[...] += jnp.concatenate([s, sq], axis=0)


def _scatter_kernel(idx_ref, pts_ref, w1_ref, b1_ref, w2_ref, b2_ref, w3_ref, b3_ref,
                    out_ref, f_ref, a0, a1, a2, a3):
    j = pl.program_id(1)
    accs = (a0, a1, a2, a3)

    @pl.when(j == 0)
    def _():
        for a in accs:
            a[...] = jnp.zeros_like(a)

    h1 = jnp.maximum(jnp.dot(pts_ref[...], w1_ref[...], preferred_element_type=jnp.float32)
                     + b1_ref[...], 0.0)
    h2 = jnp.maximum(jnp.dot(h1, w2_ref[...], preferred_element_type=jnp.float32)
                     + b2_ref[...], 0.0)
    f_ref[...] = jnp.dot(h2, w3_ref[...], preferred_element_type=jnp.float32) + b3_ref[...]

    def body(g, carry):
        base = pl.multiple_of(g * 8, 8)
        chunk = f_ref[pl.ds(base, 8), :]
        for half in range(2):
            vs, vals = [], []
            for k in range(4):
                p = half * 4 + k
                v = idx_ref[0, 0, g * 8 + p]
                vs.append(v)
                vals.append(jnp.maximum(accs[k][v, 0, :], chunk[p]))
            for k in range(4):
                accs[k][vs[k], 0, :] = vals[k]
        return carry

    jax.lax.fori_loop(0, _PCHUNK // 8, body, 0)

    @pl.when(j == _HALF - 1)
    def _():
        def cb(t, carry):
            blk = pl.ds(t * 64, 64)
            out_ref[blk, :, :] = jnp.maximum(
                jnp.maximum(a0[blk, :, :], a1[blk, :, :]),
                jnp.maximum(a2[blk, :, :], a3[blk, :, :]))
            return carry
        jax.lax.fori_loop(0, _GX * _GY // 64, cb, 0)


def _bn_fold(w, b, s, n, g, beta):
    mean = s[0] / n
    var = s[1] / n - mean * mean
    a = g * jax.lax.rsqrt(var + _EPS)
    c = beta - mean * a
    return w * a[None, :], b * a + c


def _lidar_bev_pallas(lidar_points, params):
    pts = lidar_points.reshape(-1, 7)
    xyz = (pts[:, :3] - jnp.array(_PC_LO, jnp.float32)) / jnp.array(_VOX, jnp.float32)
    pts_n = jnp.concatenate([xyz, pts[:, 3:], jnp.zeros((_NPTS, 1), jnp.float32)], axis=1)
    gx = jnp.clip(xyz[:, 0].astype(jnp.int32), 0, _GX - 1)
    gy = jnp.clip(xyz[:, 1].astype(jnp.int32), 0, _GY - 1)
    flat = (gx * _GY + gy).astype(jnp.int32).reshape(_NCHUNK, 1, _PCHUNK)

    m = params["mlp"]
    w1 = jnp.concatenate([m["w1"], jnp.zeros((1, 64), jnp.float32)], axis=0)
    b1 = m["b1"][None, :]

    n = jnp.float32(_NPTS)
    stats1 = pl.pallas_call(
        _stats1_kernel,
        grid=(_NCHUNK,),
        in_specs=[
            pl.BlockSpec((_PCHUNK, 8), lambda j: (j, 0)),
            pl.BlockSpec((8, 64), lambda j: (0, 0)),
            pl.BlockSpec((1, 64), lambda j: (0, 0)),
        ],
        out_specs=pl.BlockSpec((2, 64), lambda j: (0, 0)),
        out_shape=jax.ShapeDtypeStruct((2, 64), jnp.float32),
        compiler_params=pltpu.CompilerParams(
            dimension_semantics=("arbitrary",)),
        name="lidar_stats1",
    )(pts_n, w1, b1)

    w1f, b1f = _bn_fold(w1, b1[0], stats1, n, m["g1"], m["bb1"])
    b1f = b1f[None, :]
    w2 = m["w2"]
    b2 = m["b2"][None, :]

    stats2 = pl.pallas_call(
        _stats2_kernel,
        grid=(_NCHUNK,),
        in_specs=[
            pl.BlockSpec((_PCHUNK, 8), lambda j: (j, 0)),
            pl.BlockSpec((8, 64), lambda j: (0, 0)),
            pl.BlockSpec((1, 64), lambda j: (0, 0)),
            pl.BlockSpec((64, 128), lambda j: (0, 0)),
            pl.BlockSpec((1, 128), lambda j: (0, 0)),
        ],
        out_specs=pl.BlockSpec((2, 128), lambda j: (0, 0)),
        out_shape=jax.ShapeDtypeStruct((2, 128), jnp.float32),
        compiler_params=pltpu.CompilerParams(
            dimension_semantics=("arbitrary",)),
        name="lidar_stats2",
    )(pts_n, w1f, b1f, w2, b2)

    w2f, b2f = _bn_fold(w2, b2[0], stats2, n, m["g2"], m["bb2"])
    b2f = b2f[None, :]
    w3 = m["w3"]
    b3 = m["b3"][None, :]

    bev2 = pl.pallas_call(
        _scatter_kernel,
        grid=(2, _HALF),
        in_specs=[
            pl.BlockSpec((1, 1, _PCHUNK), lambda i, j: (i * _HALF + j, 0, 0),
                         memory_space=pltpu.SMEM),
            pl.BlockSpec((_PCHUNK, 8), lambda i, j: (i * _HALF + j, 0)),
            pl.BlockSpec((8, 64), lambda i, j: (0, 0)),
            pl.BlockSpec((1, 64), lambda i, j: (0, 0)),
            pl.BlockSpec((64, 128), lambda i, j: (0, 0)),
            pl.BlockSpec((1, 128), lambda i, j: (0, 0)),
            pl.BlockSpec((128, 256), lambda i, j: (0, 0)),
            pl.BlockSpec((1, 256), lambda i, j: (0, 0)),
        ],
        out_specs=pl.BlockSpec((_GX * _GY, 1, _D), lambda i, j: (i, 0, 0)),
        out_shape=jax.ShapeDtypeStruct((2 * _GX * _GY, 1, _D), jnp.float32),
        scratch_shapes=[pltpu.VMEM((_PCHUNK, _D), jnp.float32)]
        + [pltpu.VMEM((_GX * _GY, 1, _D), jnp.float32) for _ in range(4)],
        compiler_params=pltpu.CompilerParams(
            dimension_semantics=("parallel", "arbitrary"),
            vmem_limit_bytes=50 * 1024 * 1024),
        name="lidar_mlp_scatter",
    )(flat, pts_n, w1f, b1f, w2f, b2f, w3, b3)

    bev = jnp.maximum(bev2[:_GX * _GY, 0, :], bev2[_GX * _GY:, 0, :])
    return bev


_QB = 128
_NQB = _GX * _GY // _QB
_KT = 512
_NKT = _GX * _GY // _KT


def _ln_apply(x, g, b):
    m = x.mean(-1, keepdims=True)
    v = ((x - m) * (x - m)).mean(-1, keepdims=True)
    return g * (x - m) / jnp.sqrt(v + _EPS) + b


def _attn_kernel(qblk_ref, kvT_ref,
                 wkv1_ref, wkv2_ref,
                 wqh1_ref, qbh1_ref, vbh1_ref, owh1_ref, ob1_ref,
                 ln1g1_ref, ln1b1_ref, f1w1_ref, f1b1_ref, f2w1_ref, f2b1_ref,
                 ln2g1_ref, ln2b1_ref,
                 wqh2_ref, qbh2_ref, vbh2_ref, owh2_ref, ob2_ref,
                 ln1g2_ref, ln1b2_ref, f1w2_ref, f1b2_ref, f2w2_ref, f2b2_ref,
                 ln2g2_ref, ln2b2_ref,
                 outw_ref, outb_ref,
                 out_ref,
                 kvp1, kvp2, f_scr):
    j = pl.program_id(1)

    @pl.when(j == 0)
    def _():
        for wkv_ref, kvp in ((wkv1_ref, kvp1), (wkv2_ref, kvp2)):
            for half in range(2):
                for t in range(_NKT):
                    sl = slice(t * _KT, (t + 1) * _KT)
                    kv_t = kvT_ref[:, sl]
                    prj = jnp.dot(wkv_ref[pl.ds(half * 256, 256), :], kv_t,
                                  preferred_element_type=jnp.float32).astype(jnp.bfloat16)
                    for h in range(8):
                        kvp[half * 8 + h, :, sl] = prj[h * 32:(h + 1) * 32, :]

    scale = jnp.float32(1.0 / np.sqrt(32.0))
    x = qblk_ref[...]
    layers = (
        (wqh1_ref, qbh1_ref, vbh1_ref, owh1_ref, ob1_ref, kvp1,
         ln1g1_ref, ln1b1_ref, f1w1_ref, f1b1_ref, f2w1_ref, f2b1_ref,
         ln2g1_ref, ln2b1_ref),
        (wqh2_ref, qbh2_ref, vbh2_ref, owh2_ref, ob2_ref, kvp2,
         ln1g2_ref, ln1b2_ref, f1w2_ref, f1b2_ref, f2w2_ref, f2b2_ref,
         ln2g2_ref, ln2b2_ref),
    )
    for (wqh, qbh, vbh, owh, ob, kvp,
         ln1g, ln1b, f1w, f1b, f2w, f2b, ln2g, ln2b) in layers:
        attn = jnp.zeros((_QB, _D), jnp.float32)
        for h in range(8):
            qh = (jnp.dot(x, wqh[h], preferred_element_type=jnp.float32)
                  + qbh[h]) * scale
            qhb = qh.astype(jnp.bfloat16)
            m = jnp.full((_QB, 1), -jnp.inf, jnp.float32)
            lsum = jnp.zeros((_QB, 1), jnp.float32)
            acc = jnp.zeros((_QB, 32), jnp.float32)
            for t in range(_NKT):
                sl = slice(t * _KT, (t + 1) * _KT)
                s_t = jnp.dot(qhb, kvp[h, :, sl], preferred_element_type=jnp.float32)
                m_new = jnp.maximum(m, jnp.max(s_t, axis=1, keepdims=True))
                r = jnp.exp(m - m_new)
                p = jnp.exp(s_t - m_new)
                lsum = lsum * r + jnp.sum(p, axis=1, keepdims=True)
                acc = acc * r + jax.lax.dot_general(
                    p.astype(jnp.bfloat16), kvp[8 + h, :, sl],
                    (((1,), (1,)), ((), ())),
                    preferred_element_type=jnp.float32)
                m = m_new
            o_h = acc / lsum + vbh[h]
            attn = attn + jnp.dot(o_h, owh[h], preferred_element_type=jnp.float32)
        x = _ln_apply(x + attn + ob[...], ln1g[...], ln1b[...])
        for t in range(2):
            sl = slice(t * 512, (t + 1) * 512)
            f_scr[:, sl] = jnp.maximum(
                jnp.dot(x, f1w[:, sl], preferred_element_type=jnp.float32)
                + f1b[:, sl], 0.0)
        y = (jnp.dot(f_scr[:, :512], f2w[pl.ds(0, 512), :],
                     preferred_element_type=jnp.float32)
             + jnp.dot(f_scr[:, 512:], f2w[pl.ds(512, 512), :],
                       preferred_element_type=jnp.float32)
             + f2b[...])
        x = _ln_apply(x + y, ln2g[...], ln2b[...])

    out_ref[...] = jnp.dot(x, outw_ref[...], preferred_element_type=jnp.float32) + outb_ref[...]


def _attn_pallas(q, kvT, params):
    largs = []
    wkvs = []
    for lp in params["layers"]:
        inw = lp["in_w"]
        wqh = inw[:_D].T.reshape(_D, 8, 32).transpose(1, 0, 2)
        qbh = lp["in_b"][:_D].reshape(8, 1, 32)
        vbh = lp["in_b"][2 * _D:].reshape(8, 1, 32)
        owh = lp["out_w"].T.reshape(8, 32, _D)
        ob = lp["out_b"][None, :]
        wkvs.append(inw[_D:])
        largs += [wqh, qbh, vbh, owh, ob,
                  lp["ln1g"][None, :], lp["ln1b"][None, :],
                  lp["f1w"], lp["f1b"][None, :], lp["f2w"], lp["f2b"][None, :],
                  lp["ln2g"][None, :], lp["ln2b"][None, :]]
    outw = params["out_w"][:, :, 0, 0].T
    outb = params["out_b"][None, :]

    cspec = lambda shp: pl.BlockSpec(shp, lambda i, j: tuple(0 for _ in shp))
    in_specs = [
        pl.BlockSpec((_QB, _D), lambda i, j: (i * (_NQB // 2) + j, 0)),
        cspec((_D, _GX * _GY)),
        cspec((2 * _D, _D)), cspec((2 * _D, _D)),
    ]
    for _ in range(2):
        in_specs += [cspec((8, _D, 32)), cspec((8, 1, 32)), cspec((8, 1, 32)),
                     cspec((8, 32, _D)), cspec((1, _D)),
                     cspec((1, _D)), cspec((1, _D)),
                     cspec((_D, 4 * _D)), cspec((1, 4 * _D)),
                     cspec((4 * _D, _D)), cspec((1, _D)),
                     cspec((1, _D)), cspec((1, _D))]
    in_specs += [cspec((_D, _D)), cspec((1, _D))]

    return pl.pallas_call(
        _attn_kernel,
        grid=(2, _NQB // 2),
        in_specs=in_specs,
        out_specs=pl.BlockSpec((_QB, _D), lambda i, j: (i * (_NQB // 2) + j, 0)),
        out_shape=jax.ShapeDtypeStruct((_GX * _GY, _D), jnp.float32),
        scratch_shapes=[
            pltpu.VMEM((16, 32, _GX * _GY), jnp.bfloat16),
            pltpu.VMEM((16, 32, _GX * _GY), jnp.bfloat16),
            pltpu.VMEM((_QB, 4 * _D), jnp.float32),
        ],
        compiler_params=pltpu.CompilerParams(
            dimension_semantics=("parallel", "arbitrary"),
            vmem_limit_bytes=55 * 1024 * 1024),
        name="fused_cross_attn",
    )(q, kvT, wkvs[0], wkvs[1], *largs, outw, outb)


def kernel(images, lidar_points, params):
    B, N, C, H, W = images.shape
    x = _resnet34(images.reshape(B * N, C, H, W), params)
    imf = _conv2d(x, params["proj_w"], params["proj_b"])
    _, Cf, Hf, Wf = imf.shape
    image_features = imf.reshape(B, N, Cf, Hf, Wf)
    image_bev = _conv2d(imf, params["img_w"], params["img_b"]).reshape(B, N, Cf, Hf, Wf).mean(1)

    bev = _lidar_bev_pallas(lidar_points, params)
    lidar_bev = _conv2d(bev.reshape(1, _GX, _GY, _D).transpose(0, 3, 1, 2),
                        params["lid_w"], params["lid_b"])

    image_bev = jax.image.resize(image_bev, (B, Cf, _GX, _GY), method="bilinear")
    q = image_bev.reshape(Cf, _GX * _GY).T
    kvT = lidar_bev.reshape(Cf, _GX * _GY)
    out = _attn_pallas(q, kvT, params)
    bev_features = out.T.reshape(B, Cf, _GX, _GY)
    return bev_features, image_features, lidar_bev


# speedup vs baseline: 1.2393x; 1.2393x over previous
import jax
import jax.numpy as jnp
import numpy as np
from jax.experimental import pallas as pl
from jax.experimental.pallas import tpu as pltpu

_GX = _GY = 64
_VOX = (1.6, 1.6, 0.6)
_PC_LO = (-51.2, -51.2, -3.0)
_D = 256
_HEADS = 8
_EPS = 1e-5

_NPTS = 120000
_PCHUNK = 1000
_NCHUNK = _NPTS // _PCHUNK
_HALF = _NCHUNK // 2


def _conv2d(x, w, b=None, stride=1, pad=0):
    y = jax.lax.conv_general_dilated(x, w, (stride, stride), ((pad, pad), (pad, pad)),
                                     dimension_numbers=("NCHW", "OIHW", "NCHW"))
    if b is not None:
        y = y + b[None, :, None, None]
    return y


def _bn2d(x, g, b):
    m = x.mean((0, 2, 3), keepdims=True)
    v = x.var((0, 2, 3), keepdims=True)
    return g[None, :, None, None] * (x - m) / jnp.sqrt(v + _EPS) + b[None, :, None, None]


def _layernorm(x, g, b):
    m = x.mean(-1, keepdims=True)
    v = x.var(-1, keepdims=True)
    return g * (x - m) / jnp.sqrt(v + _EPS) + b


def _maxpool3x3s2(x):
    return jax.lax.reduce_window(x, -jnp.inf, jax.lax.max, (1, 1, 3, 3), (1, 1, 2, 2),
                                 ((0, 0), (0, 0), (1, 1), (1, 1)))


def _basic_block(x, bp, stride):
    y = jax.nn.relu(_bn2d(_conv2d(x, bp["w1"], stride=stride, pad=1), bp["g1"], bp["b1"]))
    y = _bn2d(_conv2d(y, bp["w2"], stride=1, pad=1), bp["g2"], bp["b2"])
    if "wd" in bp:
        idn = _bn2d(_conv2d(x, bp["wd"], stride=stride), bp["gd"], bp["bd"])
    else:
        idn = x
    return jax.nn.relu(y + idn)


def _resnet34(x, p):
    x = jax.nn.relu(_bn2d(_conv2d(x, p["stem"]["w"], stride=2, pad=3), p["stem"]["g"], p["stem"]["b"]))
    x = _maxpool3x3s2(x)
    for stage, s0 in zip(p["stages"], (1, 2, 2, 2)):
        for i, bp in enumerate(stage):
            x = _basic_block(x, bp, s0 if i == 0 else 1)
    return x


def _mha(q, kv, p):
    B, Sq, Dm = q.shape
    h = _HEADS
    dh = Dm // h
    qp = q @ p["in_w"][:Dm].T + p["in_b"][:Dm]
    kp = kv @ p["in_w"][Dm:2 * Dm].T + p["in_b"][Dm:2 * Dm]
    vp = kv @ p["in_w"][2 * Dm:].T + p["in_b"][2 * Dm:]
    sp = lambda t: t.reshape(B, -1, h, dh).transpose(0, 2, 1, 3)
    qh, kh, vh = sp(qp), sp(kp), sp(vp)
    a = jax.nn.softmax(jnp.einsum("bhqd,bhkd->bhqk", qh, kh) / np.sqrt(dh).astype(np.float32), axis=-1)
    o = jnp.einsum("bhqk,bhkd->bhqd", a, vh).transpose(0, 2, 1, 3).reshape(B, Sq, Dm)
    return o @ p["out_w"].T + p["out_b"]


def _cross_layer(q, kv, p):
    x = _layernorm(q + _mha(q, kv, p), p["ln1g"], p["ln1b"])
    f = jax.nn.relu(x @ p["f1w"] + p["f1b"]) @ p["f2w"] + p["f2b"]
    return _layernorm(x + f, p["ln2g"], p["ln2b"])


def _stats1_kernel(pts_ref, w_ref, b_ref, out_ref):
    j = pl.program_id(0)

    @pl.when(j == 0)
    def _():
        out_ref[...] = jnp.zeros_like(out_ref)

    h = jnp.dot(pts_ref[...], w_ref[...], preferred_element_type=jnp.float32) + b_ref[...]
    s = jnp.sum(h, axis=0, keepdims=True)
    sq = jnp.sum(h * h, axis=0, keepdims=True)
    out_ref[...] += jnp.concatenate([s, sq], axis=0)


def _stats2_kernel(pts_ref, w1_ref, b1_ref, w2_ref, b2_ref, out_ref):
    j = pl.program_id(0)

    @pl.when(j == 0)
    def _():
        out_ref[...] = jnp.zeros_like(out_ref)

    h1 = jnp.maximum(jnp.dot(pts_ref[...], w1_ref[...], preferred_element_type=jnp.float32)
                     + b1_ref[...], 0.0)
    h2 = jnp.dot(h1, w2_ref[...], preferred_element_type=jnp.float32) + b2_ref[...]
    s = jnp.sum(h2, axis=0, keepdims=True)
    sq = jnp.sum(h2 * h2, axis=0, keepdims=True)
    out_ref[...] += jnp.concatenate([s, sq], axis=0)


def _scatter_kernel(idx_ref, pts_ref, w1_ref, b1_ref, w2_ref, b2_ref, w3_ref, b3_ref,
                    out_ref, f_ref, a0, a1, a2, a3):
    j = pl.program_id(1)
    accs = (a0, a1, a2, a3)

    @pl.when(j == 0)
    def _():
        for a in accs:
            a[...] = jnp.zeros_like(a)

    h1 = jnp.maximum(jnp.dot(pts_ref[...], w1_ref[...], preferred_element_type=jnp.float32)
                     + b1_ref[...], 0.0)
    h2 = jnp.maximum(jnp.dot(h1, w2_ref[...], preferred_element_type=jnp.float32)
                     + b2_ref[...], 0.0)
    f_ref[...] = jnp.dot(h2, w3_ref[...], preferred_element_type=jnp.float32) + b3_ref[...]

    def body(g, carry):
        base = pl.multiple_of(g * 8, 8)
        chunk = f_ref[pl.ds(base, 8), :]
        for half in range(2):
            vs, vals = [], []
            for k in range(4):
                p = half * 4 + k
                v = idx_ref[0, 0, g * 8 + p]
                vs.append(v)
                vals.append(jnp.maximum(accs[k][v, 0, :], chunk[p]))
            for k in range(4):
                accs[k][vs[k], 0, :] = vals[k]
        return carry

    jax.lax.fori_loop(0, _PCHUNK // 8, body, 0)

    @pl.when(j == _HALF - 1)
    def _():
        def cb(t, carry):
            blk = pl.ds(t * 64, 64)
            out_ref[blk, :, :] = jnp.maximum(
                jnp.maximum(a0[blk, :, :], a1[blk, :, :]),
                jnp.maximum(a2[blk, :, :], a3[blk, :, :]))
            return carry
        jax.lax.fori_loop(0, _GX * _GY // 64, cb, 0)


def _bn_fold(w, b, s, n, g, beta):
    mean = s[0] / n
    var = s[1] / n - mean * mean
    a = g * jax.lax.rsqrt(var + _EPS)
    c = beta - mean * a
    return w * a[None, :], b * a + c


def _lidar_bev_pallas(lidar_points, params):
    pts = lidar_points.reshape(-1, 7)
    xyz = (pts[:, :3] - jnp.array(_PC_LO, jnp.float32)) / jnp.array(_VOX, jnp.float32)
    pts_n = jnp.concatenate([xyz, pts[:, 3:], jnp.zeros((_NPTS, 1), jnp.float32)], axis=1)
    gx = jnp.clip(xyz[:, 0].astype(jnp.int32), 0, _GX - 1)
    gy = jnp.clip(xyz[:, 1].astype(jnp.int32), 0, _GY - 1)
    flat = (gx * _GY + gy).astype(jnp.int32).reshape(_NCHUNK, 1, _PCHUNK)

    m = params["mlp"]
    w1 = jnp.concatenate([m["w1"], jnp.zeros((1, 64), jnp.float32)], axis=0)
    b1 = m["b1"][None, :]

    n = jnp.float32(_NPTS)
    stats1 = pl.pallas_call(
        _stats1_kernel,
        grid=(_NCHUNK,),
        in_specs=[
            pl.BlockSpec((_PCHUNK, 8), lambda j: (j, 0)),
            pl.BlockSpec((8, 64), lambda j: (0, 0)),
            pl.BlockSpec((1, 64), lambda j: (0, 0)),
        ],
        out_specs=pl.BlockSpec((2, 64), lambda j: (0, 0)),
        out_shape=jax.ShapeDtypeStruct((2, 64), jnp.float32),
        compiler_params=pltpu.CompilerParams(
            dimension_semantics=("arbitrary",)),
        name="lidar_stats1",
    )(pts_n, w1, b1)

    w1f, b1f = _bn_fold(w1, b1[0], stats1, n, m["g1"], m["bb1"])
    b1f = b1f[None, :]
    w2 = m["w2"]
    b2 = m["b2"][None, :]

    stats2 = pl.pallas_call(
        _stats2_kernel,
        grid=(_NCHUNK,),
        in_specs=[
            pl.BlockSpec((_PCHUNK, 8), lambda j: (j, 0)),
            pl.BlockSpec((8, 64), lambda j: (0, 0)),
            pl.BlockSpec((1, 64), lambda j: (0, 0)),
            pl.BlockSpec((64, 128), lambda j: (0, 0)),
            pl.BlockSpec((1, 128), lambda j: (0, 0)),
        ],
        out_specs=pl.BlockSpec((2, 128), lambda j: (0, 0)),
        out_shape=jax.ShapeDtypeStruct((2, 128), jnp.float32),
        compiler_params=pltpu.CompilerParams(
            dimension_semantics=("arbitrary",)),
        name="lidar_stats2",
    )(pts_n, w1f, b1f, w2, b2)

    w2f, b2f = _bn_fold(w2, b2[0], stats2, n, m["g2"], m["bb2"])
    b2f = b2f[None, :]
    w3 = m["w3"]
    b3 = m["b3"][None, :]

    bev2 = pl.pallas_call(
        _scatter_kernel,
        grid=(2, _HALF),
        in_specs=[
            pl.BlockSpec((1, 1, _PCHUNK), lambda i, j: (i * _HALF + j, 0, 0),
                         memory_space=pltpu.SMEM),
            pl.BlockSpec((_PCHUNK, 8), lambda i, j: (i * _HALF + j, 0)),
            pl.BlockSpec((8, 64), lambda i, j: (0, 0)),
            pl.BlockSpec((1, 64), lambda i, j: (0, 0)),
            pl.BlockSpec((64, 128), lambda i, j: (0, 0)),
            pl.BlockSpec((1, 128), lambda i, j: (0, 0)),
            pl.BlockSpec((128, 256), lambda i, j: (0, 0)),
            pl.BlockSpec((1, 256), lambda i, j: (0, 0)),
        ],
        out_specs=pl.BlockSpec((_GX * _GY, 1, _D), lambda i, j: (i, 0, 0)),
        out_shape=jax.ShapeDtypeStruct((2 * _GX * _GY, 1, _D), jnp.float32),
        scratch_shapes=[pltpu.VMEM((_PCHUNK, _D), jnp.float32)]
        + [pltpu.VMEM((_GX * _GY, 1, _D), jnp.float32) for _ in range(4)],
        compiler_params=pltpu.CompilerParams(
            dimension_semantics=("parallel", "arbitrary"),
            vmem_limit_bytes=50 * 1024 * 1024),
        name="lidar_mlp_scatter",
    )(flat, pts_n, w1f, b1f, w2f, b2f, w3, b3)

    bev = jnp.maximum(bev2[:_GX * _GY, 0, :], bev2[_GX * _GY:, 0, :])
    return bev


_QB = 128
_NQB = _GX * _GY // _QB
_KT = 512
_NKT = _GX * _GY // _KT


def _ln_apply(x, g, b):
    m = x.mean(-1, keepdims=True)
    v = ((x - m) * (x - m)).mean(-1, keepdims=True)
    return g * (x - m) / jnp.sqrt(v + _EPS) + b


def _attn_kernel(qblk_ref, kvT_ref,
                 wkv1_ref, wkv2_ref,
                 wqh1_ref, qbh1_ref, vbh1_ref, owh1_ref, ob1_ref,
                 ln1g1_ref, ln1b1_ref, f1w1_ref, f1b1_ref, f2w1_ref, f2b1_ref,
                 ln2g1_ref, ln2b1_ref,
                 wqh2_ref, qbh2_ref, vbh2_ref, owh2_ref, ob2_ref,
                 ln1g2_ref, ln1b2_ref, f1w2_ref, f1b2_ref, f2w2_ref, f2b2_ref,
                 ln2g2_ref, ln2b2_ref,
                 outw_ref, outb_ref,
                 out_ref,
                 kvp1, kvp2, s_ref, f_scr):
    j = pl.program_id(1)

    @pl.when(j == 0)
    def _():
        for wkv_ref, kvp in ((wkv1_ref, kvp1), (wkv2_ref, kvp2)):
            for half in range(2):
                for t in range(_NKT):
                    sl = slice(t * _KT, (t + 1) * _KT)
                    kv_t = kvT_ref[:, sl]
                    prj = jnp.dot(wkv_ref[pl.ds(half * 256, 256), :], kv_t,
                                  preferred_element_type=jnp.float32).astype(jnp.bfloat16)
                    for h in range(8):
                        kvp[half * 8 + h, :, sl] = prj[h * 32:(h + 1) * 32, :]

    scale = jnp.float32(1.0 / np.sqrt(32.0))
    x = qblk_ref[...]
    layers = (
        (wqh1_ref, qbh1_ref, vbh1_ref, owh1_ref, ob1_ref, kvp1,
         ln1g1_ref, ln1b1_ref, f1w1_ref, f1b1_ref, f2w1_ref, f2b1_ref,
         ln2g1_ref, ln2b1_ref),
        (wqh2_ref, qbh2_ref, vbh2_ref, owh2_ref, ob2_ref, kvp2,
         ln1g2_ref, ln1b2_ref, f1w2_ref, f1b2_ref, f2w2_ref, f2b2_ref,
         ln2g2_ref, ln2b2_ref),
    )
    for (wqh, qbh, vbh, owh, ob, kvp,
         ln1g, ln1b, f1w, f1b, f2w, f2b, ln2g, ln2b) in layers:
        attn = jnp.zeros((_QB, _D), jnp.float32)
        for h in range(8):
            qh = (jnp.dot(x, wqh[h], preferred_element_type=jnp.float32)
                  + qbh[h]) * scale
            qhb = qh.astype(jnp.bfloat16)
            m = jnp.full((_QB, 1), -jnp.inf, jnp.float32)
            for t in range(_NKT):
                sl = slice(t * _KT, (t + 1) * _KT)
                s_t = jnp.dot(qhb, kvp[h, :, sl], preferred_element_type=jnp.float32)
                s_ref[:, sl] = s_t
                m = jnp.maximum(m, jnp.max(s_t, axis=1, keepdims=True))
            lsum = jnp.zeros((_QB, 1), jnp.float32)
            acc = jnp.zeros((_QB, 32), jnp.float32)
            for t in range(_NKT):
                sl = slice(t * _KT, (t + 1) * _KT)
                p = jnp.exp(s_ref[:, sl] - m)
                lsum = lsum + jnp.sum(p, axis=1, keepdims=True)
                acc = acc + jax.lax.dot_general(
                    p.astype(jnp.bfloat16), kvp[8 + h, :, sl],
                    (((1,), (1,)), ((), ())),
                    preferred_element_type=jnp.float32)
            o_h = acc / lsum + vbh[h]
            attn = attn + jnp.dot(o_h, owh[h], preferred_element_type=jnp.float32)
        x = _ln_apply(x + attn + ob[...], ln1g[...], ln1b[...])
        for t in range(2):
            sl = slice(t * 512, (t + 1) * 512)
            f_scr[:, sl] = jnp.maximum(
                jnp.dot(x, f1w[:, sl], preferred_element_type=jnp.float32)
                + f1b[:, sl], 0.0)
        y = (jnp.dot(f_scr[:, :512], f2w[pl.ds(0, 512), :],
                     preferred_element_type=jnp.float32)
             + jnp.dot(f_scr[:, 512:], f2w[pl.ds(512, 512), :],
                       preferred_element_type=jnp.float32)
             + f2b[...])
        x = _ln_apply(x + y, ln2g[...], ln2b[...])

    out_ref[...] = jnp.dot(x, outw_ref[...], preferred_element_type=jnp.float32) + outb_ref[...]


def _attn_pallas(q, kvT, params):
    largs = []
    wkvs = []
    for lp in params["layers"]:
        inw = lp["in_w"]
        wqh = inw[:_D].T.reshape(_D, 8, 32).transpose(1, 0, 2)
        qbh = lp["in_b"][:_D].reshape(8, 1, 32)
        vbh = lp["in_b"][2 * _D:].reshape(8, 1, 32)
        owh = lp["out_w"].T.reshape(8, 32, _D)
        ob = lp["out_b"][None, :]
        wkvs.append(inw[_D:])
        largs += [wqh, qbh, vbh, owh, ob,
                  lp["ln1g"][None, :], lp["ln1b"][None, :],
                  lp["f1w"], lp["f1b"][None, :], lp["f2w"], lp["f2b"][None, :],
                  lp["ln2g"][None, :], lp["ln2b"][None, :]]
    outw = params["out_w"][:, :, 0, 0].T
    outb = params["out_b"][None, :]

    cspec = lambda shp: pl.BlockSpec(shp, lambda i, j: tuple(0 for _ in shp))
    in_specs = [
        pl.BlockSpec((_QB, _D), lambda i, j: (i * (_NQB // 2) + j, 0)),
        cspec((_D, _GX * _GY)),
        cspec((2 * _D, _D)), cspec((2 * _D, _D)),
    ]
    for _ in range(2):
        in_specs += [cspec((8, _D, 32)), cspec((8, 1, 32)), cspec((8, 1, 32)),
                     cspec((8, 32, _D)), cspec((1, _D)),
                     cspec((1, _D)), cspec((1, _D)),
                     cspec((_D, 4 * _D)), cspec((1, 4 * _D)),
                     cspec((4 * _D, _D)), cspec((1, _D)),
                     cspec((1, _D)), cspec((1, _D))]
    in_specs += [cspec((_D, _D)), cspec((1, _D))]

    return pl.pallas_call(
        _attn_kernel,
        grid=(2, _NQB // 2),
        in_specs=in_specs,
        out_specs=pl.BlockSpec((_QB, _D), lambda i, j: (i * (_NQB // 2) + j, 0)),
        out_shape=jax.ShapeDtypeStruct((_GX * _GY, _D), jnp.float32),
        scratch_shapes=[
            pltpu.VMEM((16, 32, _GX * _GY), jnp.bfloat16),
            pltpu.VMEM((16, 32, _GX * _GY), jnp.bfloat16),
            pltpu.VMEM((_QB, _GX * _GY), jnp.float32),
            pltpu.VMEM((_QB, 4 * _D), jnp.float32),
        ],
        compiler_params=pltpu.CompilerParams(
            dimension_semantics=("parallel", "arbitrary"),
            vmem_limit_bytes=55 * 1024 * 1024),
        name="fused_cross_attn",
    )(q, kvT, wkvs[0], wkvs[1], *largs, outw, outb)


def kernel(images, lidar_points, params):
    B, N, C, H, W = images.shape
    x = _resnet34(images.reshape(B * N, C, H, W), params)
    imf = _conv2d(x, params["proj_w"], params["proj_b"])
    _, Cf, Hf, Wf = imf.shape
    image_features = imf.reshape(B, N, Cf, Hf, Wf)
    image_bev = _conv2d(imf, params["img_w"], params["img_b"]).reshape(B, N, Cf, Hf, Wf).mean(1)

    bev = _lidar_bev_pallas(lidar_points, params)
    lidar_bev = _conv2d(bev.reshape(1, _GX, _GY, _D).transpose(0, 3, 1, 2),
                        params["lid_w"], params["lid_b"])

    image_bev = jax.image.resize(image_bev, (B, Cf, _GX, _GY), method="bilinear")
    q = image_bev.reshape(Cf, _GX * _GY).T
    kvT = lidar_bev.reshape(Cf, _GX * _GY)
    out = _attn_pallas(q, kvT, params)
    bev_features = out.T.reshape(B, Cf, _GX, _GY)
    return bev_features, image_features, lidar_bev


# speedup vs baseline: 1.2442x; 1.0040x over previous
import jax
import jax.numpy as jnp
import numpy as np
from jax.experimental import pallas as pl
from jax.experimental.pallas import tpu as pltpu

_GX = _GY = 64
_VOX = (1.6, 1.6, 0.6)
_PC_LO = (-51.2, -51.2, -3.0)
_D = 256
_HEADS = 8
_EPS = 1e-5

_NPTS = 120000
_PCHUNK = 1000
_NCHUNK = _NPTS // _PCHUNK
_HALF = _NCHUNK // 2


def _conv2d(x, w, b=None, stride=1, pad=0):
    y = jax.lax.conv_general_dilated(x, w, (stride, stride), ((pad, pad), (pad, pad)),
                                     dimension_numbers=("NCHW", "OIHW", "NCHW"))
    if b is not None:
        y = y + b[None, :, None, None]
    return y


def _bn2d(x, g, b):
    m = x.mean((0, 2, 3), keepdims=True)
    v = x.var((0, 2, 3), keepdims=True)
    return g[None, :, None, None] * (x - m) / jnp.sqrt(v + _EPS) + b[None, :, None, None]


def _layernorm(x, g, b):
    m = x.mean(-1, keepdims=True)
    v = x.var(-1, keepdims=True)
    return g * (x - m) / jnp.sqrt(v + _EPS) + b


def _maxpool3x3s2(x):
    return jax.lax.reduce_window(x, -jnp.inf, jax.lax.max, (1, 1, 3, 3), (1, 1, 2, 2),
                                 ((0, 0), (0, 0), (1, 1), (1, 1)))


def _basic_block(x, bp, stride):
    y = jax.nn.relu(_bn2d(_conv2d(x, bp["w1"], stride=stride, pad=1), bp["g1"], bp["b1"]))
    y = _bn2d(_conv2d(y, bp["w2"], stride=1, pad=1), bp["g2"], bp["b2"])
    if "wd" in bp:
        idn = _bn2d(_conv2d(x, bp["wd"], stride=stride), bp["gd"], bp["bd"])
    else:
        idn = x
    return jax.nn.relu(y + idn)


def _resnet34(x, p):
    x = jax.nn.relu(_bn2d(_conv2d(x, p["stem"]["w"], stride=2, pad=3), p["stem"]["g"], p["stem"]["b"]))
    x = _maxpool3x3s2(x)
    for stage, s0 in zip(p["stages"], (1, 2, 2, 2)):
        for i, bp in enumerate(stage):
            x = _basic_block(x, bp, s0 if i == 0 else 1)
    return x


def _mha(q, kv, p):
    B, Sq, Dm = q.shape
    h = _HEADS
    dh = Dm // h
    qp = q @ p["in_w"][:Dm].T + p["in_b"][:Dm]
    kp = kv @ p["in_w"][Dm:2 * Dm].T + p["in_b"][Dm:2 * Dm]
    vp = kv @ p["in_w"][2 * Dm:].T + p["in_b"][2 * Dm:]
    sp = lambda t: t.reshape(B, -1, h, dh).transpose(0, 2, 1, 3)
    qh, kh, vh = sp(qp), sp(kp), sp(vp)
    a = jax.nn.softmax(jnp.einsum("bhqd,bhkd->bhqk", qh, kh) / np.sqrt(dh).astype(np.float32), axis=-1)
    o = jnp.einsum("bhqk,bhkd->bhqd", a, vh).transpose(0, 2, 1, 3).reshape(B, Sq, Dm)
    return o @ p["out_w"].T + p["out_b"]


def _cross_layer(q, kv, p):
    x = _layernorm(q + _mha(q, kv, p), p["ln1g"], p["ln1b"])
    f = jax.nn.relu(x @ p["f1w"] + p["f1b"]) @ p["f2w"] + p["f2b"]
    return _layernorm(x + f, p["ln2g"], p["ln2b"])


def _stats1_kernel(pts_ref, w_ref, b_ref, out_ref):
    j = pl.program_id(0)

    @pl.when(j == 0)
    def _():
        out_ref[...] = jnp.zeros_like(out_ref)

    h = jnp.dot(pts_ref[...], w_ref[...], preferred_element_type=jnp.float32) + b_ref[...]
    s = jnp.sum(h, axis=0, keepdims=True)
    sq = jnp.sum(h * h, axis=0, keepdims=True)
    out_ref[...] += jnp.concatenate([s, sq], axis=0)


def _stats2_kernel(pts_ref, w1_ref, b1_ref, w2_ref, b2_ref, out_ref):
    j = pl.program_id(0)

    @pl.when(j == 0)
    def _():
        out_ref[...] = jnp.zeros_like(out_ref)

    h1 = jnp.maximum(jnp.dot(pts_ref[...], w1_ref[...], preferred_element_type=jnp.float32)
                     + b1_ref[...], 0.0)
    h2 = jnp.dot(h1, w2_ref[...], preferred_element_type=jnp.float32) + b2_ref[...]
    s = jnp.sum(h2, axis=0, keepdims=True)
    sq = jnp.sum(h2 * h2, axis=0, keepdims=True)
    out_ref[...] += jnp.concatenate([s, sq], axis=0)


def _scatter_kernel(idx_ref, pts_ref, w1_ref, b1_ref, w2_ref, b2_ref, w3_ref, b3_ref,
                    out_ref, f_ref, a0, a1, a2, a3):
    j = pl.program_id(1)
    accs = (a0, a1, a2, a3)

    @pl.when(j == 0)
    def _():
        for a in accs:
            a[...] = jnp.zeros_like(a)

    h1 = jnp.maximum(jnp.dot(pts_ref[...], w1_ref[...], preferred_element_type=jnp.float32)
                     + b1_ref[...], 0.0)
    h2 = jnp.maximum(jnp.dot(h1, w2_ref[...], preferred_element_type=jnp.float32)
                     + b2_ref[...], 0.0)
    f_ref[...] = jnp.dot(h2, w3_ref[...], preferred_element_type=jnp.float32) + b3_ref[...]

    def body(g, carry):
        base = pl.multiple_of(g * 8, 8)
        chunk = f_ref[pl.ds(base, 8), :]
        for half in range(2):
            vs, vals = [], []
            for k in range(4):
                p = half * 4 + k
                v = idx_ref[0, 0, g * 8 + p]
                vs.append(v)
                vals.append(jnp.maximum(accs[k][v, 0, :], chunk[p]))
            for k in range(4):
                accs[k][vs[k], 0, :] = vals[k]
        return carry

    jax.lax.fori_loop(0, _PCHUNK // 8, body, 0)

    @pl.when(j == _HALF - 1)
    def _():
        def cb(t, carry):
            blk = pl.ds(t * 64, 64)
            out_ref[blk, :, :] = jnp.maximum(
                jnp.maximum(a0[blk, :, :], a1[blk, :, :]),
                jnp.maximum(a2[blk, :, :], a3[blk, :, :]))
            return carry
        jax.lax.fori_loop(0, _GX * _GY // 64, cb, 0)


def _bn_fold(w, b, s, n, g, beta):
    mean = s[0] / n
    var = s[1] / n - mean * mean
    a = g * jax.lax.rsqrt(var + _EPS)
    c = beta - mean * a
    return w * a[None, :], b * a + c


def _lidar_bev_pallas(lidar_points, params):
    pts = lidar_points.reshape(-1, 7)
    xyz = (pts[:, :3] - jnp.array(_PC_LO, jnp.float32)) / jnp.array(_VOX, jnp.float32)
    pts_n = jnp.concatenate([xyz, pts[:, 3:], jnp.zeros((_NPTS, 1), jnp.float32)], axis=1)
    gx = jnp.clip(xyz[:, 0].astype(jnp.int32), 0, _GX - 1)
    gy = jnp.clip(xyz[:, 1].astype(jnp.int32), 0, _GY - 1)
    flat = (gx * _GY + gy).astype(jnp.int32).reshape(_NCHUNK, 1, _PCHUNK)

    m = params["mlp"]
    w1 = jnp.concatenate([m["w1"], jnp.zeros((1, 64), jnp.float32)], axis=0)
    b1 = m["b1"][None, :]

    n = jnp.float32(_NPTS)
    stats1 = pl.pallas_call(
        _stats1_kernel,
        grid=(_NCHUNK,),
        in_specs=[
            pl.BlockSpec((_PCHUNK, 8), lambda j: (j, 0)),
            pl.BlockSpec((8, 64), lambda j: (0, 0)),
            pl.BlockSpec((1, 64), lambda j: (0, 0)),
        ],
        out_specs=pl.BlockSpec((2, 64), lambda j: (0, 0)),
        out_shape=jax.ShapeDtypeStruct((2, 64), jnp.float32),
        compiler_params=pltpu.CompilerParams(
            dimension_semantics=("arbitrary",)),
        name="lidar_stats1",
    )(pts_n, w1, b1)

    w1f, b1f = _bn_fold(w1, b1[0], stats1, n, m["g1"], m["bb1"])
    b1f = b1f[None, :]
    w2 = m["w2"]
    b2 = m["b2"][None, :]

    stats2 = pl.pallas_call(
        _stats2_kernel,
        grid=(_NCHUNK,),
        in_specs=[
            pl.BlockSpec((_PCHUNK, 8), lambda j: (j, 0)),
            pl.BlockSpec((8, 64), lambda j: (0, 0)),
            pl.BlockSpec((1, 64), lambda j: (0, 0)),
            pl.BlockSpec((64, 128), lambda j: (0, 0)),
            pl.BlockSpec((1, 128), lambda j: (0, 0)),
        ],
        out_specs=pl.BlockSpec((2, 128), lambda j: (0, 0)),
        out_shape=jax.ShapeDtypeStruct((2, 128), jnp.float32),
        compiler_params=pltpu.CompilerParams(
            dimension_semantics=("arbitrary",)),
        name="lidar_stats2",
    )(pts_n, w1f, b1f, w2, b2)

    w2f, b2f = _bn_fold(w2, b2[0], stats2, n, m["g2"], m["bb2"])
    b2f = b2f[None, :]
    w3 = m["w3"]
    b3 = m["b3"][None, :]

    bev2 = pl.pallas_call(
        _scatter_kernel,
        grid=(2, _HALF),
        in_specs=[
            pl.BlockSpec((1, 1, _PCHUNK), lambda i, j: (i * _HALF + j, 0, 0),
                         memory_space=pltpu.SMEM),
            pl.BlockSpec((_PCHUNK, 8), lambda i, j: (i * _HALF + j, 0)),
            pl.BlockSpec((8, 64), lambda i, j: (0, 0)),
            pl.BlockSpec((1, 64), lambda i, j: (0, 0)),
            pl.BlockSpec((64, 128), lambda i, j: (0, 0)),
            pl.BlockSpec((1, 128), lambda i, j: (0, 0)),
            pl.BlockSpec((128, 256), lambda i, j: (0, 0)),
            pl.BlockSpec((1, 256), lambda i, j: (0, 0)),
        ],
        out_specs=pl.BlockSpec((_GX * _GY, 1, _D), lambda i, j: (i, 0, 0)),
        out_shape=jax.ShapeDtypeStruct((2 * _GX * _GY, 1, _D), jnp.float32),
        scratch_shapes=[pltpu.VMEM((_PCHUNK, _D), jnp.float32)]
        + [pltpu.VMEM((_GX * _GY, 1, _D), jnp.float32) for _ in range(4)],
        compiler_params=pltpu.CompilerParams(
            dimension_semantics=("parallel", "arbitrary"),
            vmem_limit_bytes=50 * 1024 * 1024),
        name="lidar_mlp_scatter",
    )(flat, pts_n, w1f, b1f, w2f, b2f, w3, b3)

    bev = jnp.maximum(bev2[:_GX * _GY, 0, :], bev2[_GX * _GY:, 0, :])
    return bev


_QB = 128
_NQB = _GX * _GY // _QB
_KT = 512
_NKT = _GX * _GY // _KT


def _ln_apply(x, g, b):
    m = x.mean(-1, keepdims=True)
    v = ((x - m) * (x - m)).mean(-1, keepdims=True)
    return g * (x - m) / jnp.sqrt(v + _EPS) + b


def _attn_kernel(qblk_ref, kvT_ref,
                 wkv1_ref, wkv2_ref,
                 wqh1_ref, qbh1_ref, vbh1_ref, owh1_ref, ob1_ref,
                 ln1g1_ref, ln1b1_ref, f1w1_ref, f1b1_ref, f2w1_ref, f2b1_ref,
                 ln2g1_ref, ln2b1_ref,
                 wqh2_ref, qbh2_ref, vbh2_ref, owh2_ref, ob2_ref,
                 ln1g2_ref, ln1b2_ref, f1w2_ref, f1b2_ref, f2w2_ref, f2b2_ref,
                 ln2g2_ref, ln2b2_ref,
                 outw_ref, outb_ref,
                 out_ref,
                 kvp1, kvp2, s_ref, f_scr):
    j = pl.program_id(1)

    @pl.when(j == 0)
    def _():
        for wkv_ref, kvp in ((wkv1_ref, kvp1), (wkv2_ref, kvp2)):
            for half in range(2):
                for t in range(_NKT):
                    sl = slice(t * _KT, (t + 1) * _KT)
                    kv_t = kvT_ref[:, sl]
                    prj = jnp.dot(wkv_ref[pl.ds(half * 256, 256), :], kv_t,
                                  preferred_element_type=jnp.float32)
                    for h in range(8):
                        kvp[half * 8 + h, :, sl] = prj[h * 32:(h + 1) * 32, :]

    scale = jnp.float32(1.0 / np.sqrt(32.0))
    x = qblk_ref[...]
    layers = (
        (wqh1_ref, qbh1_ref, vbh1_ref, owh1_ref, ob1_ref, kvp1,
         ln1g1_ref, ln1b1_ref, f1w1_ref, f1b1_ref, f2w1_ref, f2b1_ref,
         ln2g1_ref, ln2b1_ref),
        (wqh2_ref, qbh2_ref, vbh2_ref, owh2_ref, ob2_ref, kvp2,
         ln1g2_ref, ln1b2_ref, f1w2_ref, f1b2_ref, f2w2_ref, f2b2_ref,
         ln2g2_ref, ln2b2_ref),
    )
    for (wqh, qbh, vbh, owh, ob, kvp,
         ln1g, ln1b, f1w, f1b, f2w, f2b, ln2g, ln2b) in layers:
        attn = jnp.zeros((_QB, _D), jnp.float32)
        for h in range(8):
            qh = (jnp.dot(x, wqh[h], preferred_element_type=jnp.float32)
                  + qbh[h]) * scale
            m = jnp.full((_QB, 1), -jnp.inf, jnp.float32)
            for t in range(_NKT):
                sl = slice(t * _KT, (t + 1) * _KT)
                s_t = jnp.dot(qh, kvp[h, :, sl], preferred_element_type=jnp.float32)
                s_ref[:, sl] = s_t
                m = jnp.maximum(m, jnp.max(s_t, axis=1, keepdims=True))
            lsum = jnp.zeros((_QB, 1), jnp.float32)
            acc = jnp.zeros((_QB, 32), jnp.float32)
            for t in range(_NKT):
                sl = slice(t * _KT, (t + 1) * _KT)
                p = jnp.exp(s_ref[:, sl] - m)
                lsum = lsum + jnp.sum(p, axis=1, keepdims=True)
                acc = acc + jax.lax.dot_general(
                    p, kvp[8 + h, :, sl], (((1,), (1,)), ((), ())),
                    preferred_element_type=jnp.float32)
            o_h = acc / lsum + vbh[h]
            attn = attn + jnp.dot(o_h, owh[h], preferred_element_type=jnp.float32)
        x = _ln_apply(x + attn + ob[...], ln1g[...], ln1b[...])
        for t in range(2):
            sl = slice(t * 512, (t + 1) * 512)
            f_scr[:, sl] = jnp.maximum(
                jnp.dot(x, f1w[:, sl], preferred_element_type=jnp.float32)
                + f1b[:, sl], 0.0)
        y = (jnp.dot(f_scr[:, :512], f2w[pl.ds(0, 512), :],
                     preferred_element_type=jnp.float32)
             + jnp.dot(f_scr[:, 512:], f2w[pl.ds(512, 512), :],
                       preferred_element_type=jnp.float32)
             + f2b[...])
        x = _ln_apply(x + y, ln2g[...], ln2b[...])

    out_ref[...] = jnp.dot(x, outw_ref[...], preferred_element_type=jnp.float32) + outb_ref[...]


def _attn_pallas(q, kvT, params):
    largs = []
    wkvs = []
    for lp in params["layers"]:
        inw = lp["in_w"]
        wqh = inw[:_D].T.reshape(_D, 8, 32).transpose(1, 0, 2)
        qbh = lp["in_b"][:_D].reshape(8, 1, 32)
        vbh = lp["in_b"][2 * _D:].reshape(8, 1, 32)
        owh = lp["out_w"].T.reshape(8, 32, _D)
        ob = lp["out_b"][None, :]
        wkvs.append(inw[_D:])
        largs += [wqh, qbh, vbh, owh, ob,
                  lp["ln1g"][None, :], lp["ln1b"][None, :],
                  lp["f1w"], lp["f1b"][None, :], lp["f2w"], lp["f2b"][None, :],
                  lp["ln2g"][None, :], lp["ln2b"][None, :]]
    outw = params["out_w"][:, :, 0, 0].T
    outb = params["out_b"][None, :]

    cspec = lambda shp: pl.BlockSpec(shp, lambda i, j: tuple(0 for _ in shp))
    in_specs = [
        pl.BlockSpec((_QB, _D), lambda i, j: (i * (_NQB // 2) + j, 0)),
        cspec((_D, _GX * _GY)),
        cspec((2 * _D, _D)), cspec((2 * _D, _D)),
    ]
    for _ in range(2):
        in_specs += [cspec((8, _D, 32)), cspec((8, 1, 32)), cspec((8, 1, 32)),
                     cspec((8, 32, _D)), cspec((1, _D)),
                     cspec((1, _D)), cspec((1, _D)),
                     cspec((_D, 4 * _D)), cspec((1, 4 * _D)),
                     cspec((4 * _D, _D)), cspec((1, _D)),
                     cspec((1, _D)), cspec((1, _D))]
    in_specs += [cspec((_D, _D)), cspec((1, _D))]

    return pl.pallas_call(
        _attn_kernel,
        grid=(2, _NQB // 2),
        in_specs=in_specs,
        out_specs=pl.BlockSpec((_QB, _D), lambda i, j: (i * (_NQB // 2) + j, 0)),
        out_shape=jax.ShapeDtypeStruct((_GX * _GY, _D), jnp.float32),
        scratch_shapes=[
            pltpu.VMEM((16, 32, _GX * _GY), jnp.float32),
            pltpu.VMEM((16, 32, _GX * _GY), jnp.float32),
            pltpu.VMEM((_QB, _GX * _GY), jnp.float32),
            pltpu.VMEM((_QB, 4 * _D), jnp.float32),
        ],
        compiler_params=pltpu.CompilerParams(
            dimension_semantics=("parallel", "arbitrary"),
            vmem_limit_bytes=55 * 1024 * 1024),
        name="fused_cross_attn",
    )(q, kvT, wkvs[0], wkvs[1], *largs, outw, outb)


def kernel(images, lidar_points, params):
    B, N, C, H, W = images.shape
    x = _resnet34(images.reshape(B * N, C, H, W), params)
    imf = _conv2d(x, params["proj_w"], params["proj_b"])
    _, Cf, Hf, Wf = imf.shape
    image_features = imf.reshape(B, N, Cf, Hf, Wf)
    image_bev = _conv2d(imf, params["img_w"], params["img_b"]).reshape(B, N, Cf, Hf, Wf).mean(1)

    bev = _lidar_bev_pallas(lidar_points, params)
    lidar_bev = _conv2d(bev.reshape(1, _GX, _GY, _D).transpose(0, 3, 1, 2),
                        params["lid_w"], params["lid_b"])

    image_bev = jax.image.resize(image_bev, (B, Cf, _GX, _GY), method="bilinear")
    q = image_bev.reshape(Cf, _GX * _GY).T
    kvT = lidar_bev.reshape(Cf, _GX * _GY)
    out = _attn_pallas(q, kvT, params)
    bev_features = out.T.reshape(B, Cf, _GX, _GY)
    return bev_features, image_features, lidar_bev
